# Initial kernel scaffold; baseline (speedup 1.0000x reference)
import jax
import jax.numpy as jnp
from jax.experimental import pallas as pl


def kernel(x, edge_index, Wl1, b1, Wr1, Wl2, b2, Wr2):
    raise NotImplementedError("write your pallas kernel here")


# speedup vs baseline: 4.9324x; 4.9324x over previous
import functools

import jax
from jax import lax
import jax.numpy as jnp
from jax.experimental import pallas as pl
from jax.experimental.pallas import tpu as pltpu
from jax.experimental.pallas import tpu_sc as plsc

_N = 10000
_E = 320000
_DIN = 128
_DH = 64
_DOUT = 64

_W = 128
_NTILES = 32
_EPT = _E // _NTILES
_CB = 80
_NCH = _EPT // _CB
_NPAD = 10240
_RPT = _NPAD // 16
_ZR = 128


@functools.cache
def _make_sc_segsum():
  mesh = plsc.VectorSubcoreMesh(core_axis_name="c", subcore_axis_name="s")

  @functools.partial(
      pl.kernel,
      out_type=jax.ShapeDtypeStruct((2, _NPAD, _W), jnp.float32),
      mesh=mesh,
      scratch_types=[
          pltpu.VMEM((_CB,), jnp.int32),
          pltpu.VMEM((_CB,), jnp.int32),
          pltpu.VMEM((_CB, _W), jnp.float32),
          pltpu.VMEM((_ZR, _W), jnp.float32),
          pltpu.VMEM_SHARED((_NPAD, _W), jnp.float32),
          pltpu.SemaphoreType.DMA,
      ],
  )
  def sc(table, srci, dsti, out, srcv, dstv, rows, zb, acc, sem):
    cid = lax.axis_index("c")
    sid = lax.axis_index("s")

    zvec = jnp.zeros((16,), jnp.float32)

    def zrow(i, carry):
      for j in range(_W // 16):
        zb[i, pl.ds(j * 16, 16)] = zvec
      return carry

    lax.fori_loop(0, _ZR, zrow, 0)
    row0 = sid * _RPT
    for k in range(_RPT // _ZR):
      pltpu.sync_copy(zb, acc.at[pl.ds(row0 + k * _ZR, _ZR)])
    plsc.subcore_barrier()

    eb = (sid * 2 + cid) * _EPT

    def body(i, carry):
      base = eb + i * _CB
      pltpu.sync_copy(srci.at[pl.ds(base, _CB)], srcv)
      pltpu.async_copy(table.at[srcv], rows, sem).wait()
      pltpu.sync_copy(dsti.at[pl.ds(base, _CB)], dstv)
      pltpu.sync_copy(rows, acc.at[dstv], add=True)
      return carry

    lax.fori_loop(0, _NCH, body, 0)
    plsc.subcore_barrier()

    for k in range(_RPT // _ZR):
      r = row0 + k * _ZR
      pltpu.sync_copy(acc.at[pl.ds(r, _ZR)], zb)
      pltpu.sync_copy(zb, out.at[cid, pl.ds(r, _ZR)])

  return sc


def _tc_pre(x, Wl1, Wr1):
  def body(x_ref, wl_ref, wr_ref, t_ref, r_ref):
    xv = x_ref[...]
    p = jnp.dot(xv, wl_ref[...], preferred_element_type=jnp.float32)
    t_ref[...] = jnp.concatenate(
        [p, jnp.ones((_N, _W - _DH), jnp.float32)], axis=1)
    r_ref[...] = jnp.dot(xv, wr_ref[...], preferred_element_type=jnp.float32)

  return pl.pallas_call(
      body,
      out_shape=(jax.ShapeDtypeStruct((_N, _W), jnp.float32),
                 jax.ShapeDtypeStruct((_N, _DH), jnp.float32)),
  )(x, Wl1, Wr1)


def _tc_mid(aggp, r1, b1, Wl2, Wr2):
  def body(a_ref, r1_ref, b1_ref, wl_ref, wr_ref, t2_ref, r2_ref):
    acc = a_ref[0, :_N, :] + a_ref[1, :_N, :]
    cnt = acc[:, _DH:_DH + 1]
    inv = 1.0 / jnp.maximum(cnt, 1.0)
    o = acc[:, :_DH] * inv + b1_ref[...] + r1_ref[...]
    nrm = jnp.sqrt(jnp.sum(o * o, axis=1, keepdims=True))
    h = jnp.maximum(o / jnp.maximum(nrm, 1e-12), 0.0)
    p2 = jnp.dot(h, wl_ref[...], preferred_element_type=jnp.float32)
    t2_ref[...] = jnp.concatenate(
        [p2, jnp.ones((_N, _W - _DH), jnp.float32)], axis=1)
    r2_ref[...] = jnp.dot(h, wr_ref[...], preferred_element_type=jnp.float32)

  return pl.pallas_call(
      body,
      out_shape=(jax.ShapeDtypeStruct((_N, _W), jnp.float32),
                 jax.ShapeDtypeStruct((_N, _DOUT), jnp.float32)),
  )(aggp, r1, b1.reshape(1, _DH), Wl2, Wr2)


def _tc_post(aggp2, r2, b2):
  def body(a_ref, r2_ref, b2_ref, out_ref):
    acc = a_ref[0, :_N, :] + a_ref[1, :_N, :]
    cnt = acc[:, _DH:_DH + 1]
    inv = 1.0 / jnp.maximum(cnt, 1.0)
    o = acc[:, :_DH] * inv + b2_ref[...] + r2_ref[...]
    nrm = jnp.sqrt(jnp.sum(o * o, axis=1, keepdims=True))
    o = o / jnp.maximum(nrm, 1e-12)
    out_ref[...] = jnp.where(o > 0.0, o, jnp.exp(jnp.minimum(o, 0.0)) - 1.0)

  return pl.pallas_call(
      body,
      out_shape=jax.ShapeDtypeStruct((_N, _DOUT), jnp.float32),
  )(aggp2, r2, b2.reshape(1, _DOUT))


def kernel(x, edge_index, Wl1, b1, Wr1, Wl2, b2, Wr2):
  src = edge_index[0]
  dst = edge_index[1]
  table1, r1 = _tc_pre(x, Wl1, Wr1)
  aggp1 = _make_sc_segsum()(table1, src, dst)
  table2, r2 = _tc_mid(aggp1, r1, b1, Wl2, Wr2)
  aggp2 = _make_sc_segsum()(table2, src, dst)
  return _tc_post(aggp2, r2, b2)


# speedup vs baseline: 14.4286x; 2.9253x over previous
import functools

import jax
from jax import lax
import jax.numpy as jnp
from jax.experimental import pallas as pl
from jax.experimental.pallas import tpu as pltpu
from jax.experimental.pallas import tpu_sc as plsc

_N = 10000
_E = 320000
_DIN = 128
_DH = 64
_DOUT = 64

_W = 80
_NTILES = 32
_EPT = _E // _NTILES
_CB = 100
_NCH = _EPT // _CB
_NBUF = 4
_NPAD = 10240
_RPT = _NPAD // 16
_ZR = 128


@functools.cache
def _make_sc_segsum():
  mesh = plsc.VectorSubcoreMesh(core_axis_name="c", subcore_axis_name="s")

  @functools.partial(
      pl.kernel,
      out_type=jax.ShapeDtypeStruct((2, _NPAD, _W), jnp.float32),
      mesh=mesh,
      compiler_params=pltpu.CompilerParams(use_tc_tiling_on_sc=False),
      scratch_types=[
          pltpu.VMEM((_NCH, _CB), jnp.int32),
          pltpu.VMEM((_NCH, _CB), jnp.int32),
          pltpu.VMEM((_CB, _W), jnp.float32),
          pltpu.VMEM((_CB, _W), jnp.float32),
          pltpu.VMEM((_CB, _W), jnp.float32),
          pltpu.VMEM((_CB, _W), jnp.float32),
          pltpu.VMEM((_ZR, _W), jnp.float32),
          pltpu.VMEM_SHARED((_NPAD, _W), jnp.float32),
          pltpu.SemaphoreType.DMA,
          pltpu.SemaphoreType.DMA,
          pltpu.SemaphoreType.DMA,
          pltpu.SemaphoreType.DMA,
      ],
  )
  def sc(table, srci, dsti, out, srcv, dstv, r0, r1, r2, r3, zb, acc,
         s0, s1, s2, s3):
    rows = (r0, r1, r2, r3)
    sems = (s0, s1, s2, s3)
    cid = lax.axis_index("c")
    sid = lax.axis_index("s")
    wid = sid * 2 + cid

    pltpu.sync_copy(srci.at[wid], srcv)
    pltpu.sync_copy(dsti.at[wid], dstv)

    zvec = jnp.zeros((16,), jnp.float32)

    def zrow(i, carry):
      for j in range(_W // 16):
        zb[i, pl.ds(j * 16, 16)] = zvec
      return carry

    lax.fori_loop(0, _ZR, zrow, 0)
    row0 = sid * _RPT
    for k in range(_RPT // _ZR):
      pltpu.sync_copy(zb, acc.at[pl.ds(row0 + k * _ZR, _ZR)])
    plsc.subcore_barrier()

    def gather(c, b):
      pltpu.async_copy(table.at[srcv.at[c]], rows[b], sems[b])

    def scatter(c, b):
      pltpu.make_async_copy(table.at[srcv.at[c]], rows[b], sems[b]).wait()
      pltpu.sync_copy(rows[b], acc.at[dstv.at[c]], add=True)

    for b in range(_NBUF):
      gather(b, b)

    def block(j, carry):
      for b in range(_NBUF):
        c = j * _NBUF + b
        scatter(c, b)
        gather(c + _NBUF, b)
      return carry

    lax.fori_loop(0, _NCH // _NBUF - 1, block, 0)
    for b in range(_NBUF):
      scatter(_NCH - _NBUF + b, b)
    plsc.subcore_barrier()

    for k in range(_RPT // _ZR):
      r = row0 + k * _ZR
      pltpu.sync_copy(acc.at[pl.ds(r, _ZR)], zb)
      pltpu.sync_copy(zb, out.at[cid, pl.ds(r, _ZR)])

  return sc


def _tc_pre(x, Wl1, Wr1):
  def body(x_ref, wl_ref, wr_ref, t_ref, r_ref):
    xv = x_ref[...]
    p = jnp.dot(xv, wl_ref[...], preferred_element_type=jnp.float32)
    t_ref[...] = jnp.concatenate(
        [p, jnp.ones((_N, _W - _DH), jnp.float32)], axis=1)
    r_ref[...] = jnp.dot(xv, wr_ref[...], preferred_element_type=jnp.float32)

  return pl.pallas_call(
      body,
      out_shape=(jax.ShapeDtypeStruct((_N, _W), jnp.float32),
                 jax.ShapeDtypeStruct((_N, _DH), jnp.float32)),
  )(x, Wl1, Wr1)


def _tc_mid(aggp, r1, b1, Wl2, Wr2):
  def body(a_ref, r1_ref, b1_ref, wl_ref, wr_ref, t2_ref, r2_ref):
    acc = a_ref[0, :_N, :] + a_ref[1, :_N, :]
    cnt = acc[:, _DH:_DH + 1]
    inv = 1.0 / jnp.maximum(cnt, 1.0)
    o = acc[:, :_DH] * inv + b1_ref[...] + r1_ref[...]
    nrm = jnp.sqrt(jnp.sum(o * o, axis=1, keepdims=True))
    h = jnp.maximum(o / jnp.maximum(nrm, 1e-12), 0.0)
    p2 = jnp.dot(h, wl_ref[...], preferred_element_type=jnp.float32)
    t2_ref[...] = jnp.concatenate(
        [p2, jnp.ones((_N, _W - _DH), jnp.float32)], axis=1)
    r2_ref[...] = jnp.dot(h, wr_ref[...], preferred_element_type=jnp.float32)

  return pl.pallas_call(
      body,
      out_shape=(jax.ShapeDtypeStruct((_N, _W), jnp.float32),
                 jax.ShapeDtypeStruct((_N, _DOUT), jnp.float32)),
  )(aggp, r1, b1.reshape(1, _DH), Wl2, Wr2)


def _tc_post(aggp2, r2, b2):
  def body(a_ref, r2_ref, b2_ref, out_ref):
    acc = a_ref[0, :_N, :] + a_ref[1, :_N, :]
    cnt = acc[:, _DH:_DH + 1]
    inv = 1.0 / jnp.maximum(cnt, 1.0)
    o = acc[:, :_DH] * inv + b2_ref[...] + r2_ref[...]
    nrm = jnp.sqrt(jnp.sum(o * o, axis=1, keepdims=True))
    o = o / jnp.maximum(nrm, 1e-12)
    out_ref[...] = jnp.where(o > 0.0, o, jnp.exp(jnp.minimum(o, 0.0)) - 1.0)

  return pl.pallas_call(
      body,
      out_shape=jax.ShapeDtypeStruct((_N, _DOUT), jnp.float32),
  )(aggp2, r2, b2.reshape(1, _DOUT))


def kernel(x, edge_index, Wl1, b1, Wr1, Wl2, b2, Wr2):
  src = edge_index[0].reshape(_NTILES, _NCH, _CB)
  dst = edge_index[1].reshape(_NTILES, _NCH, _CB)
  table1, r1 = _tc_pre(x, Wl1, Wr1)
  aggp1 = _make_sc_segsum()(table1, src, dst)
  table2, r2 = _tc_mid(aggp1, r1, b1, Wl2, Wr2)
  aggp2 = _make_sc_segsum()(table2, src, dst)
  return _tc_post(aggp2, r2, b2)


# speedup vs baseline: 15.7143x; 1.0891x over previous
import functools

import jax
from jax import lax
import jax.numpy as jnp
from jax.experimental import pallas as pl
from jax.experimental.pallas import tpu as pltpu
from jax.experimental.pallas import tpu_sc as plsc

_N = 10000
_E = 320000
_DIN = 128
_DH = 64
_DOUT = 64

_W1 = 80
_W2 = 64
_NTILES = 32
_EPT = _E // _NTILES
_CB = 100
_NCH = _EPT // _CB
_NBUF = 4
_NPAD = 10240
_RPT = _NPAD // 16
_ZR = 128


@functools.cache
def _make_sc_segsum(W):
  mesh = plsc.VectorSubcoreMesh(core_axis_name="c", subcore_axis_name="s")

  @functools.partial(
      pl.kernel,
      out_type=jax.ShapeDtypeStruct((2, _NPAD, W), jnp.float32),
      mesh=mesh,
      compiler_params=pltpu.CompilerParams(use_tc_tiling_on_sc=False),
      scratch_types=[
          pltpu.VMEM((_NCH, _CB), jnp.int32),
          pltpu.VMEM((_NCH, _CB), jnp.int32),
          pltpu.VMEM((_CB, W), jnp.float32),
          pltpu.VMEM((_CB, W), jnp.float32),
          pltpu.VMEM((_CB, W), jnp.float32),
          pltpu.VMEM((_CB, W), jnp.float32),
          pltpu.VMEM((_ZR, W), jnp.float32),
          pltpu.VMEM_SHARED((_NPAD, W), jnp.float32),
          pltpu.SemaphoreType.DMA,
          pltpu.SemaphoreType.DMA,
          pltpu.SemaphoreType.DMA,
          pltpu.SemaphoreType.DMA,
      ],
  )
  def sc(table, ei, out, srcv, dstv, r0, r1, r2, r3, zb, acc,
         s0, s1, s2, s3):
    rows = (r0, r1, r2, r3)
    sems = (s0, s1, s2, s3)
    cid = lax.axis_index("c")
    sid = lax.axis_index("s")
    wid = sid * 2 + cid

    pltpu.sync_copy(ei.at[0, wid], srcv)
    pltpu.sync_copy(ei.at[1, wid], dstv)

    zvec = jnp.zeros((16,), jnp.float32)

    def zrow(i, carry):
      for j in range(W // 16):
        zb[i, pl.ds(j * 16, 16)] = zvec
      return carry

    lax.fori_loop(0, _ZR, zrow, 0)
    row0 = sid * _RPT
    for k in range(_RPT // _ZR):
      pltpu.sync_copy(zb, acc.at[pl.ds(row0 + k * _ZR, _ZR)])
    plsc.subcore_barrier()

    def gather(c, b):
      pltpu.async_copy(table.at[srcv.at[c]], rows[b], sems[b])

    def scatter(c, b):
      pltpu.make_async_copy(table.at[srcv.at[c]], rows[b], sems[b]).wait()
      pltpu.sync_copy(rows[b], acc.at[dstv.at[c]], add=True)

    for b in range(_NBUF):
      gather(b, b)

    def block(j, carry):
      for b in range(_NBUF):
        c = j * _NBUF + b
        scatter(c, b)
        gather(c + _NBUF, b)
      return carry

    lax.fori_loop(0, _NCH // _NBUF - 1, block, 0)
    for b in range(_NBUF):
      scatter(_NCH - _NBUF + b, b)
    plsc.subcore_barrier()

    for k in range(_RPT // _ZR):
      r = row0 + k * _ZR
      pltpu.sync_copy(acc.at[pl.ds(r, _ZR)], zb)
      pltpu.sync_copy(zb, out.at[cid, pl.ds(r, _ZR)])

  return sc


def _tc_pre(x, Wl1, Wr1):
  def body(x_ref, wl_ref, wr_ref, t_ref, r_ref):
    xv = x_ref[...]
    p = jnp.dot(xv, wl_ref[...], preferred_element_type=jnp.float32)
    t_ref[...] = jnp.concatenate(
        [p, jnp.ones((_N, _W1 - _DH), jnp.float32)], axis=1)
    r_ref[...] = jnp.dot(xv, wr_ref[...], preferred_element_type=jnp.float32)

  return pl.pallas_call(
      body,
      out_shape=(jax.ShapeDtypeStruct((_N, _W1), jnp.float32),
                 jax.ShapeDtypeStruct((_N, _DH), jnp.float32)),
  )(x, Wl1, Wr1)


def _tc_mid(aggp, r1, b1, Wl2, Wr2):
  def body(a_ref, r1_ref, b1_ref, wl_ref, wr_ref, t2_ref, r2_ref, inv_ref):
    acc = a_ref[0, :_N, :] + a_ref[1, :_N, :]
    cnt = acc[:, _DH:_DH + 1]
    inv = 1.0 / jnp.maximum(cnt, 1.0)
    o = acc[:, :_DH] * inv + b1_ref[...] + r1_ref[...]
    nrm = jnp.sqrt(jnp.sum(o * o, axis=1, keepdims=True))
    h = jnp.maximum(o / jnp.maximum(nrm, 1e-12), 0.0)
    t2_ref[...] = jnp.dot(h, wl_ref[...], preferred_element_type=jnp.float32)
    r2_ref[...] = jnp.dot(h, wr_ref[...], preferred_element_type=jnp.float32)
    inv_ref[...] = inv

  return pl.pallas_call(
      body,
      out_shape=(jax.ShapeDtypeStruct((_N, _W2), jnp.float32),
                 jax.ShapeDtypeStruct((_N, _DOUT), jnp.float32),
                 jax.ShapeDtypeStruct((_N, 1), jnp.float32)),
  )(aggp, r1, b1.reshape(1, _DH), Wl2, Wr2)


def _tc_post(aggp2, r2, inv, b2):
  def body(a_ref, r2_ref, inv_ref, b2_ref, out_ref):
    acc = a_ref[0, :_N, :] + a_ref[1, :_N, :]
    o = acc * inv_ref[...] + b2_ref[...] + r2_ref[...]
    nrm = jnp.sqrt(jnp.sum(o * o, axis=1, keepdims=True))
    o = o / jnp.maximum(nrm, 1e-12)
    out_ref[...] = jnp.where(o > 0.0, o, jnp.exp(jnp.minimum(o, 0.0)) - 1.0)

  return pl.pallas_call(
      body,
      out_shape=jax.ShapeDtypeStruct((_N, _DOUT), jnp.float32),
  )(aggp2, r2, inv, b2.reshape(1, _DOUT))


def kernel(x, edge_index, Wl1, b1, Wr1, Wl2, b2, Wr2):
  ei = edge_index.reshape(2, _NTILES, _NCH, _CB)
  table1, r1 = _tc_pre(x, Wl1, Wr1)
  aggp1 = _make_sc_segsum(_W1)(table1, ei)
  table2, r2, inv = _tc_mid(aggp1, r1, b1, Wl2, Wr2)
  aggp2 = _make_sc_segsum(_W2)(table2, ei)
  return _tc_post(aggp2, r2, inv, b2)


# speedup vs baseline: 17.4330x; 1.1094x over previous
import functools

import jax
from jax import lax
import jax.numpy as jnp
from jax.experimental import pallas as pl
from jax.experimental.pallas import tpu as pltpu
from jax.experimental.pallas import tpu_sc as plsc

_N = 10000
_E = 320000
_DIN = 128
_DH = 64
_DOUT = 64

_W1 = 96
_W2 = 64
_DT = jnp.bfloat16
_NTILES = 32
_EPT = _E // _NTILES
_CB = 100
_NCH = _EPT // _CB
_NBUF = 4
_NPAD = 10240
_RPT = _NPAD // 16
_ZR = 128


@functools.cache
def _make_sc_segsum(W):
  mesh = plsc.VectorSubcoreMesh(core_axis_name="c", subcore_axis_name="s")

  @functools.partial(
      pl.kernel,
      out_type=jax.ShapeDtypeStruct((2, _NPAD, W), _DT),
      mesh=mesh,
      compiler_params=pltpu.CompilerParams(use_tc_tiling_on_sc=False),
      scratch_types=[
          pltpu.VMEM((_NCH, _CB), jnp.int32),
          pltpu.VMEM((_NCH, _CB), jnp.int32),
          pltpu.VMEM((_CB, W), _DT),
          pltpu.VMEM((_CB, W), _DT),
          pltpu.VMEM((_CB, W), _DT),
          pltpu.VMEM((_CB, W), _DT),
          pltpu.VMEM((_ZR, W), _DT),
          pltpu.VMEM_SHARED((_NPAD, W), _DT),
          pltpu.SemaphoreType.DMA,
          pltpu.SemaphoreType.DMA,
          pltpu.SemaphoreType.DMA,
          pltpu.SemaphoreType.DMA,
      ],
  )
  def sc(table, ei, out, srcv, dstv, r0, r1, r2, r3, zb, acc,
         s0, s1, s2, s3):
    rows = (r0, r1, r2, r3)
    sems = (s0, s1, s2, s3)
    cid = lax.axis_index("c")
    sid = lax.axis_index("s")
    wid = sid * 2 + cid

    pltpu.sync_copy(ei.at[0, wid], srcv)
    pltpu.sync_copy(ei.at[1, wid], dstv)

    zvec = jnp.zeros((32,), _DT)

    def zrow(i, carry):
      for j in range(W // 32):
        zb[i, pl.ds(j * 32, 32)] = zvec
      return carry

    lax.fori_loop(0, _ZR, zrow, 0)
    row0 = sid * _RPT
    for k in range(_RPT // _ZR):
      pltpu.sync_copy(zb, acc.at[pl.ds(row0 + k * _ZR, _ZR)])
    plsc.subcore_barrier()

    def gather(c, b):
      pltpu.async_copy(table.at[srcv.at[c]], rows[b], sems[b])

    def scatter(c, b):
      pltpu.make_async_copy(table.at[srcv.at[c]], rows[b], sems[b]).wait()
      pltpu.sync_copy(rows[b], acc.at[dstv.at[c]], add=True)

    for b in range(_NBUF):
      gather(b, b)

    def block(j, carry):
      for b in range(_NBUF):
        c = j * _NBUF + b
        scatter(c, b)
        gather(c + _NBUF, b)
      return carry

    lax.fori_loop(0, _NCH // _NBUF - 1, block, 0)
    for b in range(_NBUF):
      scatter(_NCH - _NBUF + b, b)
    plsc.subcore_barrier()

    for k in range(_RPT // _ZR):
      r = row0 + k * _ZR
      pltpu.sync_copy(acc.at[pl.ds(r, _ZR)], zb)
      pltpu.sync_copy(zb, out.at[cid, pl.ds(r, _ZR)])

  return sc


def _tc_pre(x, Wl1, Wr1):
  def body(x_ref, wl_ref, wr_ref, t_ref, r_ref):
    xv = x_ref[...]
    p = jnp.dot(xv, wl_ref[...], preferred_element_type=jnp.float32)
    t_ref[...] = jnp.concatenate(
        [p.astype(_DT), jnp.ones((_N, _W1 - _DH), _DT)], axis=1)
    r_ref[...] = jnp.dot(xv, wr_ref[...], preferred_element_type=jnp.float32)

  return pl.pallas_call(
      body,
      out_shape=(jax.ShapeDtypeStruct((_N, _W1), _DT),
                 jax.ShapeDtypeStruct((_N, _DH), jnp.float32)),
  )(x, Wl1, Wr1)


def _tc_mid(aggp, r1, b1, Wl2, Wr2):
  def body(a_ref, r1_ref, b1_ref, wl_ref, wr_ref, t2_ref, r2_ref, inv_ref):
    acc = (a_ref[0, :_N, :] + a_ref[1, :_N, :]).astype(jnp.float32)
    cnt = acc[:, _DH:_DH + 1]
    inv = 1.0 / jnp.maximum(cnt, 1.0)
    o = acc[:, :_DH] * inv + b1_ref[...] + r1_ref[...]
    nrm = jnp.sqrt(jnp.sum(o * o, axis=1, keepdims=True))
    h = jnp.maximum(o / jnp.maximum(nrm, 1e-12), 0.0)
    t2_ref[...] = jnp.dot(
        h, wl_ref[...], preferred_element_type=jnp.float32).astype(_DT)
    r2_ref[...] = jnp.dot(h, wr_ref[...], preferred_element_type=jnp.float32)
    inv_ref[...] = inv

  return pl.pallas_call(
      body,
      out_shape=(jax.ShapeDtypeStruct((_N, _W2), _DT),
                 jax.ShapeDtypeStruct((_N, _DOUT), jnp.float32),
                 jax.ShapeDtypeStruct((_N, 1), jnp.float32)),
  )(aggp, r1, b1.reshape(1, _DH), Wl2, Wr2)


def _tc_post(aggp2, r2, inv, b2):
  def body(a_ref, r2_ref, inv_ref, b2_ref, out_ref):
    acc = (a_ref[0, :_N, :] + a_ref[1, :_N, :]).astype(jnp.float32)
    o = acc * inv_ref[...] + b2_ref[...] + r2_ref[...]
    nrm = jnp.sqrt(jnp.sum(o * o, axis=1, keepdims=True))
    o = o / jnp.maximum(nrm, 1e-12)
    out_ref[...] = jnp.where(o > 0.0, o, jnp.exp(jnp.minimum(o, 0.0)) - 1.0)

  return pl.pallas_call(
      body,
      out_shape=jax.ShapeDtypeStruct((_N, _DOUT), jnp.float32),
  )(aggp2, r2, inv, b2.reshape(1, _DOUT))


def kernel(x, edge_index, Wl1, b1, Wr1, Wl2, b2, Wr2):
  ei = edge_index.reshape(2, _NTILES, _NCH, _CB)
  table1, r1 = _tc_pre(x, Wl1, Wr1)
  aggp1 = _make_sc_segsum(_W1)(table1, ei)
  table2, r2, inv = _tc_mid(aggp1, r1, b1, Wl2, Wr2)
  aggp2 = _make_sc_segsum(_W2)(table2, ei)
  return _tc_post(aggp2, r2, inv, b2)
